# Initial kernel scaffold; baseline (speedup 1.0000x reference)
import functools

import jax
import jax.numpy as jnp
from jax import lax
from jax.experimental import pallas as pl
from jax.experimental.pallas import tpu as pltpu
from jax.experimental.pallas import tpu_sc as plsc

N = 10000
D = 256
DH = 128
E = 160000
NPAD = 10240
EP = 163840
EPT = EP // 16
CH = 128
NCH = EPT // CH
RPT = NPAD // 16
ZR = 40

_mesh = plsc.VectorSubcoreMesh(core_axis_name="c", subcore_axis_name="s")


def _segsum_body(xh0, xh1, gidx, sidx, out0, out1,
                 gidx_v, sidx_v, rows_v, zero_v, agg_sh):
    cid = lax.axis_index("c")
    sid = lax.axis_index("s")
    z16 = jnp.zeros((16,), jnp.float32)

    def zrow(i, _):
        zero_v[i // 8, pl.ds((i % 8) * 16, 16)] = z16
        return 0
    lax.fori_loop(0, ZR * 8, zrow, 0)

    def zcp(z, _):
        pltpu.sync_copy(zero_v, agg_sh.at[pl.ds(sid * RPT + z * ZR, ZR)])
        return 0
    lax.fori_loop(0, RPT // ZR, zcp, 0)
    plsc.subcore_barrier()

    def chunk(g, _):
        base = sid * EPT + g * CH
        pltpu.sync_copy(gidx.at[pl.ds(base, CH)], gidx_v)
        pltpu.sync_copy(sidx.at[pl.ds(base, CH)], sidx_v)

        @pl.when(cid == 0)
        def _():
            pltpu.sync_copy(xh0.at[gidx_v], rows_v)

        @pl.when(cid == 1)
        def _():
            pltpu.sync_copy(xh1.at[gidx_v], rows_v)

        pltpu.sync_copy(rows_v, agg_sh.at[sidx_v], add=True)
        return 0
    lax.fori_loop(0, NCH, chunk, 0)
    plsc.subcore_barrier()

    @pl.when(cid == 0)
    def _():
        pltpu.sync_copy(agg_sh.at[pl.ds(sid * RPT, RPT)],
                        out0.at[pl.ds(sid * RPT, RPT)])

    @pl.when(cid == 1)
    def _():
        pltpu.sync_copy(agg_sh.at[pl.ds(sid * RPT, RPT)],
                        out1.at[pl.ds(sid * RPT, RPT)])


_segsum = pl.kernel(
    _segsum_body,
    out_type=(jax.ShapeDtypeStruct((NPAD, DH), jnp.float32),
              jax.ShapeDtypeStruct((NPAD, DH), jnp.float32)),
    mesh=_mesh,
    scratch_types=[
        pltpu.VMEM((CH,), jnp.int32),
        pltpu.VMEM((CH,), jnp.int32),
        pltpu.VMEM((CH, DH), jnp.float32),
        pltpu.VMEM((ZR, DH), jnp.float32),
        pltpu.VMEM_SHARED((NPAD, DH), jnp.float32),
    ],
)


def _deg_body(idx_a, idx_b, out_a, out_b, hist_v, idx_v, rowbuf_v, out_v, stage_sh):
    cid = lax.axis_index("c")
    sid = lax.axis_index("s")
    z16 = jnp.zeros((16,), jnp.float32)
    ones16 = jnp.full((16,), 1.0, jnp.float32)

    def zh(i, _):
        hist_v[pl.ds(i * 16, 16)] = z16
        return 0
    lax.fori_loop(0, NPAD // 16, zh, 0)

    def chunk(g, _):
        base = sid * EPT + g * CH

        @pl.when(cid == 0)
        def _():
            pltpu.sync_copy(idx_a.at[pl.ds(base, CH)], idx_v)

        @pl.when(cid == 1)
        def _():
            pltpu.sync_copy(idx_b.at[pl.ds(base, CH)], idx_v)

        def inner(j, _):
            iv = idx_v[pl.ds(j * 16, 16)]
            plsc.addupdate_scatter(hist_v, [iv], ones16)
            return 0
        lax.fori_loop(0, CH // 16, inner, 0)
        return 0
    lax.fori_loop(0, NCH, chunk, 0)

    pltpu.sync_copy(hist_v, stage_sh.at[sid])
    plsc.subcore_barrier()

    def zo(j, _):
        out_v[pl.ds(j * 16, 16)] = z16
        return 0
    lax.fori_loop(0, RPT // 16, zo, 0)
    for r in range(16):
        pltpu.sync_copy(stage_sh.at[r, pl.ds(sid * RPT, RPT)], rowbuf_v)

        def addj(j, _):
            sl = pl.ds(j * 16, 16)
            out_v[sl] = out_v[sl] + rowbuf_v[sl]
            return 0
        lax.fori_loop(0, RPT // 16, addj, 0)

    @pl.when(cid == 0)
    def _():
        pltpu.sync_copy(out_v, out_a.at[pl.ds(sid * RPT, RPT)])

    @pl.when(cid == 1)
    def _():
        pltpu.sync_copy(out_v, out_b.at[pl.ds(sid * RPT, RPT)])


_deg = pl.kernel(
    _deg_body,
    out_type=(jax.ShapeDtypeStruct((NPAD,), jnp.float32),
              jax.ShapeDtypeStruct((NPAD,), jnp.float32)),
    mesh=_mesh,
    scratch_types=[
        pltpu.VMEM((NPAD,), jnp.float32),
        pltpu.VMEM((CH,), jnp.int32),
        pltpu.VMEM((RPT,), jnp.float32),
        pltpu.VMEM((RPT,), jnp.float32),
        pltpu.VMEM_SHARED((16, NPAD), jnp.float32),
    ],
)

BN = 1000


def _mm_body(split, relu, agg0, agg1, degr, x0, x1, wl, wr, b, *outs):
    rdeg = 1.0 / jnp.maximum(degr[...], 1.0)
    mean = jnp.concatenate([agg0[...] * rdeg, agg1[...] * rdeg], axis=1)
    xx = jnp.concatenate([x0[...], x1[...]], axis=1)
    dn = (((1,), (1,)), ((), ()))
    acc = lax.dot_general(mean, wl[...], dn, preferred_element_type=jnp.float32)
    acc = acc + lax.dot_general(xx, wr[...], dn, preferred_element_type=jnp.float32)
    acc = acc + b[...]
    if relu:
        acc = jnp.maximum(acc, 0.0)
    if split:
        outs[0][...] = acc[:, :DH]
        outs[1][...] = acc[:, DH:]
    else:
        outs[0][...] = acc


def _mm(agg0, agg1, degr, x0, x1, wl, wr, b, relu, split):
    row = pl.BlockSpec((BN, DH), lambda i: (i, 0))
    in_specs = [
        row, row,
        pl.BlockSpec((BN, 1), lambda i: (i, 0)),
        row, row,
        pl.BlockSpec((D, D), lambda i: (0, 0)),
        pl.BlockSpec((D, D), lambda i: (0, 0)),
        pl.BlockSpec((1, D), lambda i: (0, 0)),
    ]
    if split:
        out_shape = (jax.ShapeDtypeStruct((N, DH), jnp.float32),
                     jax.ShapeDtypeStruct((N, DH), jnp.float32))
        out_specs = (pl.BlockSpec((BN, DH), lambda i: (i, 0)),
                     pl.BlockSpec((BN, DH), lambda i: (i, 0)))
    else:
        out_shape = jax.ShapeDtypeStruct((N, D), jnp.float32)
        out_specs = pl.BlockSpec((BN, D), lambda i: (i, 0))
    return pl.pallas_call(
        functools.partial(_mm_body, split, relu),
        grid=(N // BN,),
        in_specs=in_specs,
        out_specs=out_specs,
        out_shape=out_shape,
    )(agg0, agg1, degr, x0, x1, wl, wr, b)


def kernel(s, t, edge_index,
           Wl_s0, bl_s0, Wr_s0, Wl_t0, bl_t0, Wr_t0,
           Wl_s1, bl_s1, Wr_s1, Wl_t1, bl_t1, Wr_t1):
    src = edge_index[0]
    dst = edge_index[1]
    pad_g = jnp.zeros((EP - E,), jnp.int32)
    pad_s = jnp.full((EP - E,), N, jnp.int32)
    src_g = jnp.concatenate([src, pad_g])
    src_s = jnp.concatenate([src, pad_s])
    dst_g = jnp.concatenate([dst, pad_g])
    dst_s = jnp.concatenate([dst, pad_s])

    s_h0, s_h1 = s[:, :DH], s[:, DH:]
    t_h0, t_h1 = t[:, :DH], t[:, DH:]

    deg_d_p, deg_s_p = _deg(dst_s, src_s)
    deg_d = deg_d_p[:N].reshape(N, 1)
    deg_s = deg_s_p[:N].reshape(N, 1)

    as0_0, as0_1 = _segsum(t_h0, t_h1, src_g, dst_s)
    at0_0, at0_1 = _segsum(s_h0, s_h1, dst_g, src_s)

    s1_0, s1_1 = _mm(as0_0, as0_1, deg_d, t_h0, t_h1,
                     Wl_s0, Wr_s0, bl_s0.reshape(1, D), relu=True, split=True)
    t1_0, t1_1 = _mm(at0_0, at0_1, deg_s, s_h0, s_h1,
                     Wl_t0, Wr_t0, bl_t0.reshape(1, D), relu=True, split=True)

    as1_0, as1_1 = _segsum(t1_0, t1_1, src_g, dst_s)
    at1_0, at1_1 = _segsum(s1_0, s1_1, dst_g, src_s)

    s_out = _mm(as1_0, as1_1, deg_d, t1_0, t1_1,
                Wl_s1, Wr_s1, bl_s1.reshape(1, D), relu=False, split=False)
    t_out = _mm(at1_0, at1_1, deg_s, s1_0, s1_1,
                Wl_t1, Wr_t1, bl_t1.reshape(1, D), relu=False, split=False)
    return (s_out, t_out)


# speedup vs baseline: 2.2869x; 2.2869x over previous
import functools

import jax
import jax.numpy as jnp
from jax import lax
from jax.experimental import pallas as pl
from jax.experimental.pallas import tpu as pltpu
from jax.experimental.pallas import tpu_sc as plsc

N = 10000
D = 256
DH = 128
E = 160000
NPAD = 10240
EP = 163840
EPT = EP // 16
CH = 128
NCH = EPT // CH
RPT = NPAD // 16
ZR = 40

_mesh = plsc.VectorSubcoreMesh(core_axis_name="c", subcore_axis_name="s")


def _segsum_body(xh0, xh1, gidx, sidx, out0, out1,
                 gidx_v, sidx_v, rows_v, zero_v, agg_sh):
    cid = lax.axis_index("c")
    sid = lax.axis_index("s")
    z16 = jnp.zeros((16,), jnp.float32)

    def zrow(i, _):
        zero_v[i // 8, pl.ds((i % 8) * 16, 16)] = z16
        return 0
    lax.fori_loop(0, ZR * 8, zrow, 0)

    def zcp(z, _):
        pltpu.sync_copy(zero_v, agg_sh.at[pl.ds(sid * RPT + z * ZR, ZR)])
        return 0
    lax.fori_loop(0, RPT // ZR, zcp, 0)
    plsc.subcore_barrier()

    def chunk(g, _):
        base = sid * EPT + g * CH
        pltpu.sync_copy(gidx.at[pl.ds(base, CH)], gidx_v)
        pltpu.sync_copy(sidx.at[pl.ds(base, CH)], sidx_v)

        @pl.when(cid == 0)
        def _():
            pltpu.sync_copy(xh0.at[gidx_v], rows_v)

        @pl.when(cid == 1)
        def _():
            pltpu.sync_copy(xh1.at[gidx_v], rows_v)

        pltpu.sync_copy(rows_v, agg_sh.at[sidx_v], add=True)
        return 0
    lax.fori_loop(0, NCH, chunk, 0)
    plsc.subcore_barrier()

    @pl.when(cid == 0)
    def _():
        pltpu.sync_copy(agg_sh.at[pl.ds(sid * RPT, RPT)],
                        out0.at[pl.ds(sid * RPT, RPT)])

    @pl.when(cid == 1)
    def _():
        pltpu.sync_copy(agg_sh.at[pl.ds(sid * RPT, RPT)],
                        out1.at[pl.ds(sid * RPT, RPT)])


_segsum = pl.kernel(
    _segsum_body,
    out_type=(jax.ShapeDtypeStruct((NPAD, DH), jnp.float32),
              jax.ShapeDtypeStruct((NPAD, DH), jnp.float32)),
    mesh=_mesh,
    scratch_types=[
        pltpu.VMEM((CH,), jnp.int32),
        pltpu.VMEM((CH,), jnp.int32),
        pltpu.VMEM((CH, DH), jnp.float32),
        pltpu.VMEM((ZR, DH), jnp.float32),
        pltpu.VMEM_SHARED((NPAD, DH), jnp.float32),
    ],
    compiler_params=pltpu.CompilerParams(needs_layout_passes=False),
)


def _deg_body(idx_a, idx_b, out_a, out_b, hist_v, idx_v, rowbuf_v, out_v, stage_sh):
    cid = lax.axis_index("c")
    sid = lax.axis_index("s")
    z16 = jnp.zeros((16,), jnp.float32)
    ones16 = jnp.full((16,), 1.0, jnp.float32)

    def zh(i, _):
        hist_v[pl.ds(i * 16, 16)] = z16
        return 0
    lax.fori_loop(0, NPAD // 16, zh, 0)

    def chunk(g, _):
        base = sid * EPT + g * CH

        @pl.when(cid == 0)
        def _():
            pltpu.sync_copy(idx_a.at[pl.ds(base, CH)], idx_v)

        @pl.when(cid == 1)
        def _():
            pltpu.sync_copy(idx_b.at[pl.ds(base, CH)], idx_v)

        def inner(j, _):
            iv = idx_v[pl.ds(j * 16, 16)]
            plsc.addupdate_scatter(hist_v, [iv], ones16)
            return 0
        lax.fori_loop(0, CH // 16, inner, 0)
        return 0
    lax.fori_loop(0, NCH, chunk, 0)

    pltpu.sync_copy(hist_v, stage_sh.at[sid])
    plsc.subcore_barrier()

    def zo(j, _):
        out_v[pl.ds(j * 16, 16)] = z16
        return 0
    lax.fori_loop(0, RPT // 16, zo, 0)
    for r in range(16):
        pltpu.sync_copy(stage_sh.at[r, pl.ds(sid * RPT, RPT)], rowbuf_v)

        def addj(j, _):
            sl = pl.ds(j * 16, 16)
            out_v[sl] = out_v[sl] + rowbuf_v[sl]
            return 0
        lax.fori_loop(0, RPT // 16, addj, 0)

    @pl.when(cid == 0)
    def _():
        pltpu.sync_copy(out_v, out_a.at[pl.ds(sid * RPT, RPT)])

    @pl.when(cid == 1)
    def _():
        pltpu.sync_copy(out_v, out_b.at[pl.ds(sid * RPT, RPT)])


_deg = pl.kernel(
    _deg_body,
    out_type=(jax.ShapeDtypeStruct((NPAD,), jnp.float32),
              jax.ShapeDtypeStruct((NPAD,), jnp.float32)),
    mesh=_mesh,
    scratch_types=[
        pltpu.VMEM((NPAD,), jnp.float32),
        pltpu.VMEM((CH,), jnp.int32),
        pltpu.VMEM((RPT,), jnp.float32),
        pltpu.VMEM((RPT,), jnp.float32),
        pltpu.VMEM_SHARED((16, NPAD), jnp.float32),
    ],
    compiler_params=pltpu.CompilerParams(needs_layout_passes=False),
)

BN = 1000


def _mm_body(split, relu, agg0, agg1, degr, x0, x1, wl, wr, b, *outs):
    rdeg = 1.0 / jnp.maximum(degr[...], 1.0)
    mean = jnp.concatenate([agg0[...] * rdeg, agg1[...] * rdeg], axis=1)
    xx = jnp.concatenate([x0[...], x1[...]], axis=1)
    dn = (((1,), (1,)), ((), ()))
    acc = lax.dot_general(mean, wl[...], dn, preferred_element_type=jnp.float32)
    acc = acc + lax.dot_general(xx, wr[...], dn, preferred_element_type=jnp.float32)
    acc = acc + b[...]
    if relu:
        acc = jnp.maximum(acc, 0.0)
    if split:
        outs[0][...] = acc[:, :DH]
        outs[1][...] = acc[:, DH:]
    else:
        outs[0][...] = acc


def _mm(agg0, agg1, degr, x0, x1, wl, wr, b, relu, split):
    row = pl.BlockSpec((BN, DH), lambda i: (i, 0))
    in_specs = [
        row, row,
        pl.BlockSpec((BN, 1), lambda i: (i, 0)),
        row, row,
        pl.BlockSpec((D, D), lambda i: (0, 0)),
        pl.BlockSpec((D, D), lambda i: (0, 0)),
        pl.BlockSpec((1, D), lambda i: (0, 0)),
    ]
    if split:
        out_shape = (jax.ShapeDtypeStruct((N, DH), jnp.float32),
                     jax.ShapeDtypeStruct((N, DH), jnp.float32))
        out_specs = (pl.BlockSpec((BN, DH), lambda i: (i, 0)),
                     pl.BlockSpec((BN, DH), lambda i: (i, 0)))
    else:
        out_shape = jax.ShapeDtypeStruct((N, D), jnp.float32)
        out_specs = pl.BlockSpec((BN, D), lambda i: (i, 0))
    return pl.pallas_call(
        functools.partial(_mm_body, split, relu),
        grid=(N // BN,),
        in_specs=in_specs,
        out_specs=out_specs,
        out_shape=out_shape,
    )(agg0, agg1, degr, x0, x1, wl, wr, b)


def kernel(s, t, edge_index,
           Wl_s0, bl_s0, Wr_s0, Wl_t0, bl_t0, Wr_t0,
           Wl_s1, bl_s1, Wr_s1, Wl_t1, bl_t1, Wr_t1):
    src = edge_index[0]
    dst = edge_index[1]
    pad_g = jnp.zeros((EP - E,), jnp.int32)
    pad_s = jnp.full((EP - E,), N, jnp.int32)
    src_g = jnp.concatenate([src, pad_g])
    src_s = jnp.concatenate([src, pad_s])
    dst_g = jnp.concatenate([dst, pad_g])
    dst_s = jnp.concatenate([dst, pad_s])

    s_h0, s_h1 = s[:, :DH], s[:, DH:]
    t_h0, t_h1 = t[:, :DH], t[:, DH:]

    deg_d_p, deg_s_p = _deg(dst_s, src_s)
    deg_d = deg_d_p[:N].reshape(N, 1)
    deg_s = deg_s_p[:N].reshape(N, 1)

    as0_0, as0_1 = _segsum(t_h0, t_h1, src_g, dst_s)
    at0_0, at0_1 = _segsum(s_h0, s_h1, dst_g, src_s)

    s1_0, s1_1 = _mm(as0_0, as0_1, deg_d, t_h0, t_h1,
                     Wl_s0, Wr_s0, bl_s0.reshape(1, D), relu=True, split=True)
    t1_0, t1_1 = _mm(at0_0, at0_1, deg_s, s_h0, s_h1,
                     Wl_t0, Wr_t0, bl_t0.reshape(1, D), relu=True, split=True)

    as1_0, as1_1 = _segsum(t1_0, t1_1, src_g, dst_s)
    at1_0, at1_1 = _segsum(s1_0, s1_1, dst_g, src_s)

    s_out = _mm(as1_0, as1_1, deg_d, t1_0, t1_1,
                Wl_s1, Wr_s1, bl_s1.reshape(1, D), relu=False, split=False)
    t_out = _mm(at1_0, at1_1, deg_s, s1_0, s1_1,
                Wl_t1, Wr_t1, bl_t1.reshape(1, D), relu=False, split=False)
    return (s_out, t_out)


# speedup vs baseline: 3.0543x; 1.3356x over previous
import functools

import jax
import jax.numpy as jnp
from jax import lax
from jax.experimental import pallas as pl
from jax.experimental.pallas import tpu as pltpu
from jax.experimental.pallas import tpu_sc as plsc

N = 10000
D = 256
DH = 128
E = 160000
NPAD = 10240
EP = 163840
EPT = EP // 16
CH = 128
NCH = EPT // CH
RPT = NPAD // 16
ZR = 40

_mesh = plsc.VectorSubcoreMesh(core_axis_name="c", subcore_axis_name="s")


def _segsum_body(xh0, xh1, gidx, sidx, out0, out1,
                 gi_a, si_a, gi_b, si_b, rows_a, rows_b, agg_sh,
                 gsem_a, gsem_b, isem_a, isem_b):
    cid = lax.axis_index("c")
    sid = lax.axis_index("s")
    z16 = jnp.zeros((16,), jnp.float32)

    def istart(g, gi, si, sem):
        base = sid * EPT + g * CH
        pltpu.async_copy(gidx.at[pl.ds(base, CH)], gi, sem)
        pltpu.async_copy(sidx.at[pl.ds(base, CH)], si, sem)

    def iwait(gi, sem):
        pltpu.make_async_copy(gidx.at[pl.ds(0, CH)], gi, sem).wait()
        pltpu.make_async_copy(gidx.at[pl.ds(0, CH)], gi, sem).wait()

    def gstart(gi, buf, sem):
        @pl.when(cid == 0)
        def _():
            pltpu.async_copy(xh0.at[gi], buf, sem)

        @pl.when(cid == 1)
        def _():
            pltpu.async_copy(xh1.at[gi], buf, sem)

    def gwait(buf, sem):
        pltpu.make_async_copy(xh0.at[pl.ds(0, CH)], buf, sem).wait()

    def zrow(i, _):
        rows_b[i // 8, pl.ds((i % 8) * 16, 16)] = z16
        return 0
    lax.fori_loop(0, CH * 8, zrow, 0)
    istart(0, gi_a, si_a, isem_a)
    iwait(gi_a, isem_a)
    gstart(gi_a, rows_a, gsem_a)
    istart(1, gi_b, si_b, isem_b)

    def zcp(z, _):
        pltpu.sync_copy(rows_b, agg_sh.at[pl.ds(sid * RPT + z * CH, CH)])
        return 0
    lax.fori_loop(0, RPT // CH, zcp, 0)
    plsc.subcore_barrier()

    def pair(p, _):
        g0 = p * 2
        iwait(gi_b, isem_b)
        gwait(rows_a, gsem_a)
        gstart(gi_b, rows_b, gsem_b)
        pltpu.sync_copy(rows_a, agg_sh.at[si_a], add=True)

        @pl.when(p < NCH // 2 - 1)
        def _():
            istart(g0 + 2, gi_a, si_a, isem_a)

        gwait(rows_b, gsem_b)

        @pl.when(p < NCH // 2 - 1)
        def _():
            iwait(gi_a, isem_a)
            gstart(gi_a, rows_a, gsem_a)

        pltpu.sync_copy(rows_b, agg_sh.at[si_b], add=True)

        @pl.when(p < NCH // 2 - 1)
        def _():
            istart(g0 + 3, gi_b, si_b, isem_b)
        return 0
    lax.fori_loop(0, NCH // 2, pair, 0)
    plsc.subcore_barrier()

    @pl.when(cid == 0)
    def _():
        pltpu.sync_copy(agg_sh.at[pl.ds(sid * RPT, RPT)],
                        out0.at[pl.ds(sid * RPT, RPT)])

    @pl.when(cid == 1)
    def _():
        pltpu.sync_copy(agg_sh.at[pl.ds(sid * RPT, RPT)],
                        out1.at[pl.ds(sid * RPT, RPT)])


_segsum = pl.kernel(
    _segsum_body,
    out_type=(jax.ShapeDtypeStruct((NPAD, DH), jnp.float32),
              jax.ShapeDtypeStruct((NPAD, DH), jnp.float32)),
    mesh=_mesh,
    scratch_types=[
        pltpu.VMEM((CH,), jnp.int32),
        pltpu.VMEM((CH,), jnp.int32),
        pltpu.VMEM((CH,), jnp.int32),
        pltpu.VMEM((CH,), jnp.int32),
        pltpu.VMEM((CH, DH), jnp.float32),
        pltpu.VMEM((CH, DH), jnp.float32),
        pltpu.VMEM_SHARED((NPAD, DH), jnp.float32),
        pltpu.SemaphoreType.DMA,
        pltpu.SemaphoreType.DMA,
        pltpu.SemaphoreType.DMA,
        pltpu.SemaphoreType.DMA,
    ],
    compiler_params=pltpu.CompilerParams(needs_layout_passes=False),
)


def _deg_body(idx_a, idx_b, out_a, out_b, hist_v, idx_v, rowbuf_v, out_v, stage_sh):
    cid = lax.axis_index("c")
    sid = lax.axis_index("s")
    z16 = jnp.zeros((16,), jnp.float32)
    ones16 = jnp.full((16,), 1.0, jnp.float32)

    def zh(i, _):
        hist_v[pl.ds(i * 16, 16)] = z16
        return 0
    lax.fori_loop(0, NPAD // 16, zh, 0)

    def chunk(g, _):
        base = sid * EPT + g * CH

        @pl.when(cid == 0)
        def _():
            pltpu.sync_copy(idx_a.at[pl.ds(base, CH)], idx_v)

        @pl.when(cid == 1)
        def _():
            pltpu.sync_copy(idx_b.at[pl.ds(base, CH)], idx_v)

        def inner(j, _):
            iv = idx_v[pl.ds(j * 16, 16)]
            plsc.addupdate_scatter(hist_v, [iv], ones16)
            return 0
        lax.fori_loop(0, CH // 16, inner, 0)
        return 0
    lax.fori_loop(0, NCH, chunk, 0)

    pltpu.sync_copy(hist_v, stage_sh.at[sid])
    plsc.subcore_barrier()

    def zo(j, _):
        out_v[pl.ds(j * 16, 16)] = z16
        return 0
    lax.fori_loop(0, RPT // 16, zo, 0)
    for r in range(16):
        pltpu.sync_copy(stage_sh.at[r, pl.ds(sid * RPT, RPT)], rowbuf_v)

        def addj(j, _):
            sl = pl.ds(j * 16, 16)
            out_v[sl] = out_v[sl] + rowbuf_v[sl]
            return 0
        lax.fori_loop(0, RPT // 16, addj, 0)

    @pl.when(cid == 0)
    def _():
        pltpu.sync_copy(out_v, out_a.at[pl.ds(sid * RPT, RPT)])

    @pl.when(cid == 1)
    def _():
        pltpu.sync_copy(out_v, out_b.at[pl.ds(sid * RPT, RPT)])


_deg = pl.kernel(
    _deg_body,
    out_type=(jax.ShapeDtypeStruct((NPAD,), jnp.float32),
              jax.ShapeDtypeStruct((NPAD,), jnp.float32)),
    mesh=_mesh,
    scratch_types=[
        pltpu.VMEM((NPAD,), jnp.float32),
        pltpu.VMEM((CH,), jnp.int32),
        pltpu.VMEM((RPT,), jnp.float32),
        pltpu.VMEM((RPT,), jnp.float32),
        pltpu.VMEM_SHARED((16, NPAD), jnp.float32),
    ],
    compiler_params=pltpu.CompilerParams(needs_layout_passes=False),
)

BN = 1000


def _mm_body(split, relu, agg0, agg1, degr, x0, x1, wl, wr, b, *outs):
    rdeg = 1.0 / jnp.maximum(degr[...], 1.0)
    mean = jnp.concatenate([agg0[...] * rdeg, agg1[...] * rdeg], axis=1)
    xx = jnp.concatenate([x0[...], x1[...]], axis=1)
    dn = (((1,), (1,)), ((), ()))
    acc = lax.dot_general(mean, wl[...], dn, preferred_element_type=jnp.float32)
    acc = acc + lax.dot_general(xx, wr[...], dn, preferred_element_type=jnp.float32)
    acc = acc + b[...]
    if relu:
        acc = jnp.maximum(acc, 0.0)
    if split:
        outs[0][...] = acc[:, :DH]
        outs[1][...] = acc[:, DH:]
    else:
        outs[0][...] = acc


def _mm(agg0, agg1, degr, x0, x1, wl, wr, b, relu, split):
    row = pl.BlockSpec((BN, DH), lambda i: (i, 0))
    in_specs = [
        row, row,
        pl.BlockSpec((BN, 1), lambda i: (i, 0)),
        row, row,
        pl.BlockSpec((D, D), lambda i: (0, 0)),
        pl.BlockSpec((D, D), lambda i: (0, 0)),
        pl.BlockSpec((1, D), lambda i: (0, 0)),
    ]
    if split:
        out_shape = (jax.ShapeDtypeStruct((N, DH), jnp.float32),
                     jax.ShapeDtypeStruct((N, DH), jnp.float32))
        out_specs = (pl.BlockSpec((BN, DH), lambda i: (i, 0)),
                     pl.BlockSpec((BN, DH), lambda i: (i, 0)))
    else:
        out_shape = jax.ShapeDtypeStruct((N, D), jnp.float32)
        out_specs = pl.BlockSpec((BN, D), lambda i: (i, 0))
    return pl.pallas_call(
        functools.partial(_mm_body, split, relu),
        grid=(N // BN,),
        in_specs=in_specs,
        out_specs=out_specs,
        out_shape=out_shape,
    )(agg0, agg1, degr, x0, x1, wl, wr, b)


def kernel(s, t, edge_index,
           Wl_s0, bl_s0, Wr_s0, Wl_t0, bl_t0, Wr_t0,
           Wl_s1, bl_s1, Wr_s1, Wl_t1, bl_t1, Wr_t1):
    src = edge_index[0]
    dst = edge_index[1]
    pad_g = jnp.zeros((EP - E,), jnp.int32)
    pad_s = jnp.full((EP - E,), N, jnp.int32)
    src_g = jnp.concatenate([src, pad_g])
    src_s = jnp.concatenate([src, pad_s])
    dst_g = jnp.concatenate([dst, pad_g])
    dst_s = jnp.concatenate([dst, pad_s])

    s_h0, s_h1 = s[:, :DH], s[:, DH:]
    t_h0, t_h1 = t[:, :DH], t[:, DH:]

    deg_d_p, deg_s_p = _deg(dst_s, src_s)
    deg_d = deg_d_p[:N].reshape(N, 1)
    deg_s = deg_s_p[:N].reshape(N, 1)

    as0_0, as0_1 = _segsum(t_h0, t_h1, src_g, dst_s)
    at0_0, at0_1 = _segsum(s_h0, s_h1, dst_g, src_s)

    s1_0, s1_1 = _mm(as0_0, as0_1, deg_d, t_h0, t_h1,
                     Wl_s0, Wr_s0, bl_s0.reshape(1, D), relu=True, split=True)
    t1_0, t1_1 = _mm(at0_0, at0_1, deg_s, s_h0, s_h1,
                     Wl_t0, Wr_t0, bl_t0.reshape(1, D), relu=True, split=True)

    as1_0, as1_1 = _segsum(t1_0, t1_1, src_g, dst_s)
    at1_0, at1_1 = _segsum(s1_0, s1_1, dst_g, src_s)

    s_out = _mm(as1_0, as1_1, deg_d, t1_0, t1_1,
                Wl_s1, Wr_s1, bl_s1.reshape(1, D), relu=False, split=False)
    t_out = _mm(at1_0, at1_1, deg_s, s1_0, s1_1,
                Wl_t1, Wr_t1, bl_t1.reshape(1, D), relu=False, split=False)
    return (s_out, t_out)


# speedup vs baseline: 3.6239x; 1.1865x over previous
import functools

import jax
import jax.numpy as jnp
from jax import lax
from jax.experimental import pallas as pl
from jax.experimental.pallas import tpu as pltpu
from jax.experimental.pallas import tpu_sc as plsc

N = 10000
D = 256
DH = 128
E = 160000
HALF = N // 2
DUMP = HALF
NPADH = 5248
RPTH = NPADH // 16
EP = 163840
EPT = EP // 16
HPE = EPT // 2
RCH = 128
NRC = HPE // RCH
CH2 = 64
CAP = HPE + CH2

NPAD = 10240
CHD = 128
NCHD = EPT // CHD

_mesh = plsc.VectorSubcoreMesh(core_axis_name="c", subcore_axis_name="s")


def _segsum_body(x, gidx, sidx_lo, sidx_hi, out0, out1,
                 gcomp, scomp, rg, rs, gi_a, si_a, gi_b, si_b,
                 rows_a, rows_b, acc_sh, gsem_a, gsem_b):
    cid = lax.axis_index("c")
    sid = lax.axis_index("s")
    z16 = jnp.zeros((16,), jnp.float32)
    zi16 = jnp.zeros((16,), jnp.int32)
    dump16 = jnp.full((16,), DUMP, jnp.int32)

    def zrow(i, _):
        r = i // 16
        rem = i % 16
        rows_a[r, rem // 8, pl.ds((rem % 8) * 16, 16)] = z16
        return 0
    lax.fori_loop(0, CH2 * 16, zrow, 0)

    def zcp(z, _):
        pltpu.sync_copy(rows_a, acc_sh.at[pl.ds(sid * RPTH + z * CH2, CH2)])
        return 0
    lax.fori_loop(0, RPTH // CH2, zcp, 0)
    pltpu.sync_copy(rows_a.at[pl.ds(0, RPTH % CH2)],
                    acc_sh.at[pl.ds(sid * RPTH + (RPTH // CH2) * CH2,
                                    RPTH % CH2)])
    plsc.subcore_barrier()

    def prep_idx(g, gi, si):
        for q in range(CH2 // 16):
            gi[pl.ds(q * 16, 16)] = gcomp[pl.ds(g * CH2 + q * 16, 16)]
            si[pl.ds(q * 16, 16)] = scomp[pl.ds(g * CH2 + q * 16, 16)]

    def gstart(gi, buf, sem):
        pltpu.async_copy(x.at[gi], buf, sem)

    def gwait(buf, sem):
        pltpu.make_async_copy(x.at[pl.ds(0, CH2)], buf, sem).wait()

    for h in range(2):
        def raw_chunk(rc, cnt):
            base = sid * EPT + h * HPE + rc * RCH
            pltpu.sync_copy(gidx.at[pl.ds(base, RCH)], rg)

            @pl.when(cid == 0)
            def _():
                pltpu.sync_copy(sidx_lo.at[pl.ds(base, RCH)], rs)

            @pl.when(cid == 1)
            def _():
                pltpu.sync_copy(sidx_hi.at[pl.ds(base, RCH)], rs)

            def lane(j, c):
                gv = rg[pl.ds(j * 16, 16)]
                sv = rs[pl.ds(j * 16, 16)]
                m = sv < DUMP
                plsc.store_compressed(gcomp.at[pl.ds(c, 16)], gv, mask=m)
                plsc.store_compressed(scomp.at[pl.ds(c, 16)], sv, mask=m)
                return c + jnp.sum(m.astype(jnp.int32))
            return lax.fori_loop(0, RCH // 16, lane, cnt)
        cnt = lax.fori_loop(0, NRC, raw_chunk, jnp.int32(0))

        npt = ((cnt + CH2 - 1) // CH2) * CH2
        for it in range(CH2 // 16):
            off = cnt + it * 16

            @pl.when(off < npt)
            def _():
                gcomp[pl.ds(off, 16)] = zi16
                scomp[pl.ds(off, 16)] = dump16
        nch = npt // CH2

        @pl.when(nch > 0)
        def _():
            prep_idx(0, gi_a, si_a)
            gstart(gi_a, rows_a, gsem_a)

        def chunk(g, _):
            @pl.when(g % 2 == 0)
            def _():
                @pl.when(g + 1 < nch)
                def _():
                    prep_idx(g + 1, gi_b, si_b)
                    gstart(gi_b, rows_b, gsem_b)
                gwait(rows_a, gsem_a)
                pltpu.sync_copy(rows_a, acc_sh.at[si_a], add=True)

            @pl.when(g % 2 == 1)
            def _():
                @pl.when(g + 1 < nch)
                def _():
                    prep_idx(g + 1, gi_a, si_a)
                    gstart(gi_a, rows_a, gsem_a)
                gwait(rows_b, gsem_b)
                pltpu.sync_copy(rows_b, acc_sh.at[si_b], add=True)
            return 0
        lax.fori_loop(0, nch, chunk, 0)

    plsc.subcore_barrier()

    @pl.when(cid == 0)
    def _():
        pltpu.sync_copy(acc_sh.at[pl.ds(sid * RPTH, RPTH)],
                        out0.at[pl.ds(sid * RPTH, RPTH)])

    @pl.when(cid == 1)
    def _():
        pltpu.sync_copy(acc_sh.at[pl.ds(sid * RPTH, RPTH)],
                        out1.at[pl.ds(sid * RPTH, RPTH)])


_segsum = pl.kernel(
    _segsum_body,
    out_type=(jax.ShapeDtypeStruct((NPADH, 2, DH), jnp.float32),
              jax.ShapeDtypeStruct((NPADH, 2, DH), jnp.float32)),
    mesh=_mesh,
    scratch_types=[
        pltpu.VMEM((CAP,), jnp.int32),
        pltpu.VMEM((CAP,), jnp.int32),
        pltpu.VMEM((RCH,), jnp.int32),
        pltpu.VMEM((RCH,), jnp.int32),
        pltpu.VMEM((CH2,), jnp.int32),
        pltpu.VMEM((CH2,), jnp.int32),
        pltpu.VMEM((CH2,), jnp.int32),
        pltpu.VMEM((CH2,), jnp.int32),
        pltpu.VMEM((CH2, 2, DH), jnp.float32),
        pltpu.VMEM((CH2, 2, DH), jnp.float32),
        pltpu.VMEM_SHARED((NPADH, 2, DH), jnp.float32),
        pltpu.SemaphoreType.DMA,
        pltpu.SemaphoreType.DMA,
    ],
    compiler_params=pltpu.CompilerParams(needs_layout_passes=False),
)


def _deg_body(idx_a, idx_b, out_a, out_b, hist_v, idx_v, rowbuf_v, out_v, stage_sh):
    cid = lax.axis_index("c")
    sid = lax.axis_index("s")
    z16 = jnp.zeros((16,), jnp.float32)
    ones16 = jnp.full((16,), 1.0, jnp.float32)

    def zh(i, _):
        hist_v[pl.ds(i * 16, 16)] = z16
        return 0
    lax.fori_loop(0, NPAD // 16, zh, 0)

    def chunk(g, _):
        base = sid * EPT + g * CHD

        @pl.when(cid == 0)
        def _():
            pltpu.sync_copy(idx_a.at[pl.ds(base, CHD)], idx_v)

        @pl.when(cid == 1)
        def _():
            pltpu.sync_copy(idx_b.at[pl.ds(base, CHD)], idx_v)

        def inner(j, _):
            iv = idx_v[pl.ds(j * 16, 16)]
            plsc.addupdate_scatter(hist_v, [iv], ones16)
            return 0
        lax.fori_loop(0, CHD // 16, inner, 0)
        return 0
    lax.fori_loop(0, NCHD, chunk, 0)

    pltpu.sync_copy(hist_v, stage_sh.at[sid])
    plsc.subcore_barrier()

    rpt = NPAD // 16

    def zo(j, _):
        out_v[pl.ds(j * 16, 16)] = z16
        return 0
    lax.fori_loop(0, rpt // 16, zo, 0)
    for r in range(16):
        pltpu.sync_copy(stage_sh.at[r, pl.ds(sid * rpt, rpt)], rowbuf_v)

        def addj(j, _):
            sl = pl.ds(j * 16, 16)
            out_v[sl] = out_v[sl] + rowbuf_v[sl]
            return 0
        lax.fori_loop(0, rpt // 16, addj, 0)

    @pl.when(cid == 0)
    def _():
        pltpu.sync_copy(out_v, out_a.at[pl.ds(sid * rpt, rpt)])

    @pl.when(cid == 1)
    def _():
        pltpu.sync_copy(out_v, out_b.at[pl.ds(sid * rpt, rpt)])


_deg = pl.kernel(
    _deg_body,
    out_type=(jax.ShapeDtypeStruct((NPAD,), jnp.float32),
              jax.ShapeDtypeStruct((NPAD,), jnp.float32)),
    mesh=_mesh,
    scratch_types=[
        pltpu.VMEM((NPAD,), jnp.float32),
        pltpu.VMEM((CHD,), jnp.int32),
        pltpu.VMEM((NPAD // 16,), jnp.float32),
        pltpu.VMEM((NPAD // 16,), jnp.float32),
        pltpu.VMEM_SHARED((16, NPAD), jnp.float32),
    ],
    compiler_params=pltpu.CompilerParams(needs_layout_passes=False),
)

BN = 1000


def _mm_body(relu, agg_lo, agg_hi, degr, x, wl, wr, b, out):
    i = pl.program_id(0)
    agg = jnp.where(i < 5, agg_lo[...], agg_hi[...])
    rdeg = 1.0 / jnp.maximum(degr[...], 1.0)
    mean = agg * rdeg
    dn = (((1,), (1,)), ((), ()))
    acc = lax.dot_general(mean, wl[...], dn, preferred_element_type=jnp.float32)
    acc = acc + lax.dot_general(x[...], wr[...], dn,
                                preferred_element_type=jnp.float32)
    acc = acc + b[...]
    if relu:
        acc = jnp.maximum(acc, 0.0)
    out[...] = acc


def _mm(agg_lo, agg_hi, degr, x, wl, wr, b, relu):
    in_specs = [
        pl.BlockSpec((BN, D), lambda i: (jnp.minimum(i, 4), 0)),
        pl.BlockSpec((BN, D), lambda i: (jnp.maximum(i - 5, 0), 0)),
        pl.BlockSpec((BN, 1), lambda i: (i, 0)),
        pl.BlockSpec((BN, D), lambda i: (i, 0)),
        pl.BlockSpec((D, D), lambda i: (0, 0)),
        pl.BlockSpec((D, D), lambda i: (0, 0)),
        pl.BlockSpec((1, D), lambda i: (0, 0)),
    ]
    return pl.pallas_call(
        functools.partial(_mm_body, relu),
        grid=(N // BN,),
        in_specs=in_specs,
        out_specs=pl.BlockSpec((BN, D), lambda i: (i, 0)),
        out_shape=jax.ShapeDtypeStruct((N, D), jnp.float32),
    )(agg_lo, agg_hi, degr, x, wl, wr, b)


def kernel(s, t, edge_index,
           Wl_s0, bl_s0, Wr_s0, Wl_t0, bl_t0, Wr_t0,
           Wl_s1, bl_s1, Wr_s1, Wl_t1, bl_t1, Wr_t1):
    src = edge_index[0]
    dst = edge_index[1]
    npad_e = EP - E
    pad_g = jnp.zeros((npad_e,), jnp.int32)
    pad_d = jnp.full((npad_e,), DUMP, jnp.int32)
    pad_n = jnp.full((npad_e,), N, jnp.int32)

    src_g = jnp.concatenate([src, pad_g])
    dst_g = jnp.concatenate([dst, pad_g])
    dst_lo = jnp.concatenate([jnp.where(dst < HALF, dst, DUMP), pad_d])
    dst_hi = jnp.concatenate([jnp.where(dst >= HALF, dst - HALF, DUMP), pad_d])
    src_lo = jnp.concatenate([jnp.where(src < HALF, src, DUMP), pad_d])
    src_hi = jnp.concatenate([jnp.where(src >= HALF, src - HALF, DUMP), pad_d])
    dst_f = jnp.concatenate([dst, pad_n])
    src_f = jnp.concatenate([src, pad_n])

    deg_d_p, deg_s_p = _deg(dst_f, src_f)
    deg_d = deg_d_p[:N].reshape(N, 1)
    deg_s = deg_s_p[:N].reshape(N, 1)

    t3 = t.reshape(N, 2, DH)
    s3 = s.reshape(N, 2, DH)
    as0_lo, as0_hi = _segsum(t3, src_g, dst_lo, dst_hi)
    at0_lo, at0_hi = _segsum(s3, dst_g, src_lo, src_hi)

    s1 = _mm(as0_lo.reshape(NPADH, D), as0_hi.reshape(NPADH, D), deg_d, t, Wl_s0, Wr_s0,
             bl_s0.reshape(1, D), relu=True)
    t1 = _mm(at0_lo.reshape(NPADH, D), at0_hi.reshape(NPADH, D), deg_s, s, Wl_t0, Wr_t0,
             bl_t0.reshape(1, D), relu=True)

    as1_lo, as1_hi = _segsum(t1.reshape(N, 2, DH), src_g, dst_lo, dst_hi)
    at1_lo, at1_hi = _segsum(s1.reshape(N, 2, DH), dst_g, src_lo, src_hi)

    s_out = _mm(as1_lo.reshape(NPADH, D), as1_hi.reshape(NPADH, D), deg_d, t1, Wl_s1, Wr_s1,
                bl_s1.reshape(1, D), relu=False)
    t_out = _mm(at1_lo.reshape(NPADH, D), at1_hi.reshape(NPADH, D), deg_s, s1, Wl_t1, Wr_t1,
                bl_t1.reshape(1, D), relu=False)
    return (s_out, t_out)


# speedup vs baseline: 4.6031x; 1.2702x over previous
import functools

import jax
import jax.numpy as jnp
from jax import lax
from jax.experimental import pallas as pl
from jax.experimental.pallas import tpu as pltpu
from jax.experimental.pallas import tpu_sc as plsc

N = 10000
D = 256
DH = 128
E = 160000
HALF = N // 2
DUMP = HALF
NPADH = 5248
RPTH = NPADH // 16
EP = 163840
EPT = EP // 16
HPE = EPT // 2
RCH = 512
NRC = HPE // RCH
CH2 = 64
CAP = HPE + CH2

NPAD = 10240
CHD = 128
NCHD = EPT // CHD

_mesh = plsc.VectorSubcoreMesh(core_axis_name="c", subcore_axis_name="s")


def _segsum_body(x, gidx, sidx_lo, sidx_hi, out0, out1,
                 gcomp, scomp, rg, rs, gi_a, si_a, gi_b, si_b,
                 rows_a, rows_b, acc_sh, gsem_a, gsem_b):
    cid = lax.axis_index("c")
    sid = lax.axis_index("s")
    z16 = jnp.zeros((16,), jnp.float32)
    zi16 = jnp.zeros((16,), jnp.int32)
    dump16 = jnp.full((16,), DUMP, jnp.int32)

    def zrow(i, _):
        r = i // 16
        rem = i % 16
        rows_a[r, rem // 8, pl.ds((rem % 8) * 16, 16)] = z16
        return 0
    lax.fori_loop(0, CH2 * 16, zrow, 0)

    def zcp(z, _):
        pltpu.sync_copy(rows_a, acc_sh.at[pl.ds(sid * RPTH + z * CH2, CH2)])
        return 0
    lax.fori_loop(0, RPTH // CH2, zcp, 0)
    pltpu.sync_copy(rows_a.at[pl.ds(0, RPTH % CH2)],
                    acc_sh.at[pl.ds(sid * RPTH + (RPTH // CH2) * CH2,
                                    RPTH % CH2)])
    plsc.subcore_barrier()

    def prep_idx(g, gi, si):
        for q in range(CH2 // 16):
            gi[pl.ds(q * 16, 16)] = gcomp[pl.ds(g * CH2 + q * 16, 16)]
            si[pl.ds(q * 16, 16)] = scomp[pl.ds(g * CH2 + q * 16, 16)]

    def gstart(gi, buf, sem):
        pltpu.async_copy(x.at[gi], buf, sem)

    def gwait(buf, sem):
        pltpu.make_async_copy(x.at[pl.ds(0, CH2)], buf, sem).wait()

    def rstart(h, rc, slot, sem):
        base = sid * EPT + h * HPE + rc * RCH
        pltpu.async_copy(gidx.at[pl.ds(base, RCH)], rg.at[slot], sem)

        @pl.when(cid == 0)
        def _():
            pltpu.async_copy(sidx_lo.at[pl.ds(base, RCH)], rs.at[slot], sem)

        @pl.when(cid == 1)
        def _():
            pltpu.async_copy(sidx_hi.at[pl.ds(base, RCH)], rs.at[slot], sem)

    def rwait(slot, sem):
        pltpu.make_async_copy(gidx.at[pl.ds(0, RCH)], rg.at[slot], sem).wait()
        pltpu.make_async_copy(gidx.at[pl.ds(0, RCH)], rs.at[slot], sem).wait()

    for h in range(2):
        rstart(h, 0, 0, gsem_a)

        def raw_chunk(rc, cnt):
            par = rc % 2

            @pl.when(par == 0)
            def _():
                rwait(0, gsem_a)

                @pl.when(rc + 1 < NRC)
                def _():
                    rstart(h, rc + 1, 1, gsem_b)

            @pl.when(par == 1)
            def _():
                rwait(1, gsem_b)

                @pl.when(rc + 1 < NRC)
                def _():
                    rstart(h, rc + 1, 0, gsem_a)

            def lane(j, c):
                gv = rg[par, pl.ds(j * 16, 16)]
                sv = rs[par, pl.ds(j * 16, 16)]
                m = sv < DUMP
                plsc.store_compressed(gcomp.at[pl.ds(c, 16)], gv, mask=m)
                plsc.store_compressed(scomp.at[pl.ds(c, 16)], sv, mask=m)
                return c + jnp.sum(m.astype(jnp.int32))
            return lax.fori_loop(0, RCH // 16, lane, cnt)
        cnt = lax.fori_loop(0, NRC, raw_chunk, jnp.int32(0))

        npt = ((cnt + CH2 - 1) // CH2) * CH2
        for it in range(CH2 // 16):
            off = cnt + it * 16

            @pl.when(off < npt)
            def _():
                gcomp[pl.ds(off, 16)] = zi16
                scomp[pl.ds(off, 16)] = dump16
        nch = npt // CH2

        @pl.when(nch > 0)
        def _():
            prep_idx(0, gi_a, si_a)
            gstart(gi_a, rows_a, gsem_a)

        def chunk(g, _):
            @pl.when(g % 2 == 0)
            def _():
                @pl.when(g + 1 < nch)
                def _():
                    prep_idx(g + 1, gi_b, si_b)
                    gstart(gi_b, rows_b, gsem_b)
                gwait(rows_a, gsem_a)
                pltpu.sync_copy(rows_a, acc_sh.at[si_a], add=True)

            @pl.when(g % 2 == 1)
            def _():
                @pl.when(g + 1 < nch)
                def _():
                    prep_idx(g + 1, gi_a, si_a)
                    gstart(gi_a, rows_a, gsem_a)
                gwait(rows_b, gsem_b)
                pltpu.sync_copy(rows_b, acc_sh.at[si_b], add=True)
            return 0
        lax.fori_loop(0, nch, chunk, 0)

    plsc.subcore_barrier()

    @pl.when(cid == 0)
    def _():
        pltpu.sync_copy(acc_sh.at[pl.ds(sid * RPTH, RPTH)],
                        out0.at[pl.ds(sid * RPTH, RPTH)])

    @pl.when(cid == 1)
    def _():
        pltpu.sync_copy(acc_sh.at[pl.ds(sid * RPTH, RPTH)],
                        out1.at[pl.ds(sid * RPTH, RPTH)])


_segsum = pl.kernel(
    _segsum_body,
    out_type=(jax.ShapeDtypeStruct((NPADH, 2, DH), jnp.float32),
              jax.ShapeDtypeStruct((NPADH, 2, DH), jnp.float32)),
    mesh=_mesh,
    scratch_types=[
        pltpu.VMEM((CAP,), jnp.int32),
        pltpu.VMEM((CAP,), jnp.int32),
        pltpu.VMEM((2, RCH), jnp.int32),
        pltpu.VMEM((2, RCH), jnp.int32),
        pltpu.VMEM((CH2,), jnp.int32),
        pltpu.VMEM((CH2,), jnp.int32),
        pltpu.VMEM((CH2,), jnp.int32),
        pltpu.VMEM((CH2,), jnp.int32),
        pltpu.VMEM((CH2, 2, DH), jnp.float32),
        pltpu.VMEM((CH2, 2, DH), jnp.float32),
        pltpu.VMEM_SHARED((NPADH, 2, DH), jnp.float32),
        pltpu.SemaphoreType.DMA,
        pltpu.SemaphoreType.DMA,
    ],
    compiler_params=pltpu.CompilerParams(needs_layout_passes=False),
)


def _deg_body(idx_a, idx_b, out_a, out_b, hist_v, idx_v, rowbuf_v, out_v, stage_sh):
    cid = lax.axis_index("c")
    sid = lax.axis_index("s")
    z16 = jnp.zeros((16,), jnp.float32)
    ones16 = jnp.full((16,), 1.0, jnp.float32)

    def zh(i, _):
        hist_v[pl.ds(i * 16, 16)] = z16
        return 0
    lax.fori_loop(0, NPAD // 16, zh, 0)

    def chunk(g, _):
        base = sid * EPT + g * CHD

        @pl.when(cid == 0)
        def _():
            pltpu.sync_copy(idx_a.at[pl.ds(base, CHD)], idx_v)

        @pl.when(cid == 1)
        def _():
            pltpu.sync_copy(idx_b.at[pl.ds(base, CHD)], idx_v)

        def inner(j, _):
            iv = idx_v[pl.ds(j * 16, 16)]
            plsc.addupdate_scatter(hist_v, [iv], ones16)
            return 0
        lax.fori_loop(0, CHD // 16, inner, 0)
        return 0
    lax.fori_loop(0, NCHD, chunk, 0)

    pltpu.sync_copy(hist_v, stage_sh.at[sid])
    plsc.subcore_barrier()

    rpt = NPAD // 16

    def zo(j, _):
        out_v[pl.ds(j * 16, 16)] = z16
        return 0
    lax.fori_loop(0, rpt // 16, zo, 0)
    for r in range(16):
        pltpu.sync_copy(stage_sh.at[r, pl.ds(sid * rpt, rpt)], rowbuf_v)

        def addj(j, _):
            sl = pl.ds(j * 16, 16)
            out_v[sl] = out_v[sl] + rowbuf_v[sl]
            return 0
        lax.fori_loop(0, rpt // 16, addj, 0)

    @pl.when(cid == 0)
    def _():
        pltpu.sync_copy(out_v, out_a.at[pl.ds(sid * rpt, rpt)])

    @pl.when(cid == 1)
    def _():
        pltpu.sync_copy(out_v, out_b.at[pl.ds(sid * rpt, rpt)])


_deg = pl.kernel(
    _deg_body,
    out_type=(jax.ShapeDtypeStruct((NPAD,), jnp.float32),
              jax.ShapeDtypeStruct((NPAD,), jnp.float32)),
    mesh=_mesh,
    scratch_types=[
        pltpu.VMEM((NPAD,), jnp.float32),
        pltpu.VMEM((CHD,), jnp.int32),
        pltpu.VMEM((NPAD // 16,), jnp.float32),
        pltpu.VMEM((NPAD // 16,), jnp.float32),
        pltpu.VMEM_SHARED((16, NPAD), jnp.float32),
    ],
    compiler_params=pltpu.CompilerParams(needs_layout_passes=False),
)

BN = 1000


def _mm_body(relu, agg_lo, agg_hi, degr, x, wl, wr, b, out):
    i = pl.program_id(0)
    agg = jnp.where(i < 5, agg_lo[...], agg_hi[...])
    rdeg = 1.0 / jnp.maximum(degr[...], 1.0)
    mean = agg * rdeg
    dn = (((1,), (1,)), ((), ()))
    acc = lax.dot_general(mean, wl[...], dn, preferred_element_type=jnp.float32)
    acc = acc + lax.dot_general(x[...], wr[...], dn,
                                preferred_element_type=jnp.float32)
    acc = acc + b[...]
    if relu:
        acc = jnp.maximum(acc, 0.0)
    out[...] = acc


def _mm(agg_lo, agg_hi, degr, x, wl, wr, b, relu):
    in_specs = [
        pl.BlockSpec((BN, D), lambda i: (jnp.minimum(i, 4), 0)),
        pl.BlockSpec((BN, D), lambda i: (jnp.maximum(i - 5, 0), 0)),
        pl.BlockSpec((BN, 1), lambda i: (i, 0)),
        pl.BlockSpec((BN, D), lambda i: (i, 0)),
        pl.BlockSpec((D, D), lambda i: (0, 0)),
        pl.BlockSpec((D, D), lambda i: (0, 0)),
        pl.BlockSpec((1, D), lambda i: (0, 0)),
    ]
    return pl.pallas_call(
        functools.partial(_mm_body, relu),
        grid=(N // BN,),
        in_specs=in_specs,
        out_specs=pl.BlockSpec((BN, D), lambda i: (i, 0)),
        out_shape=jax.ShapeDtypeStruct((N, D), jnp.float32),
    )(agg_lo, agg_hi, degr, x, wl, wr, b)


def kernel(s, t, edge_index,
           Wl_s0, bl_s0, Wr_s0, Wl_t0, bl_t0, Wr_t0,
           Wl_s1, bl_s1, Wr_s1, Wl_t1, bl_t1, Wr_t1):
    src = edge_index[0]
    dst = edge_index[1]
    npad_e = EP - E
    pad_g = jnp.zeros((npad_e,), jnp.int32)
    pad_d = jnp.full((npad_e,), DUMP, jnp.int32)
    pad_n = jnp.full((npad_e,), N, jnp.int32)

    src_g = jnp.concatenate([src, pad_g])
    dst_g = jnp.concatenate([dst, pad_g])
    dst_lo = jnp.concatenate([jnp.where(dst < HALF, dst, DUMP), pad_d])
    dst_hi = jnp.concatenate([jnp.where(dst >= HALF, dst - HALF, DUMP), pad_d])
    src_lo = jnp.concatenate([jnp.where(src < HALF, src, DUMP), pad_d])
    src_hi = jnp.concatenate([jnp.where(src >= HALF, src - HALF, DUMP), pad_d])
    dst_f = jnp.concatenate([dst, pad_n])
    src_f = jnp.concatenate([src, pad_n])

    deg_d_p, deg_s_p = _deg(dst_f, src_f)
    deg_d = deg_d_p[:N].reshape(N, 1)
    deg_s = deg_s_p[:N].reshape(N, 1)

    t3 = t.reshape(N, 2, DH)
    s3 = s.reshape(N, 2, DH)
    as0_lo, as0_hi = _segsum(t3, src_g, dst_lo, dst_hi)
    at0_lo, at0_hi = _segsum(s3, dst_g, src_lo, src_hi)

    s1 = _mm(as0_lo.reshape(NPADH, D), as0_hi.reshape(NPADH, D), deg_d, t, Wl_s0, Wr_s0,
             bl_s0.reshape(1, D), relu=True)
    t1 = _mm(at0_lo.reshape(NPADH, D), at0_hi.reshape(NPADH, D), deg_s, s, Wl_t0, Wr_t0,
             bl_t0.reshape(1, D), relu=True)

    as1_lo, as1_hi = _segsum(t1.reshape(N, 2, DH), src_g, dst_lo, dst_hi)
    at1_lo, at1_hi = _segsum(s1.reshape(N, 2, DH), dst_g, src_lo, src_hi)

    s_out = _mm(as1_lo.reshape(NPADH, D), as1_hi.reshape(NPADH, D), deg_d, t1, Wl_s1, Wr_s1,
                bl_s1.reshape(1, D), relu=False)
    t_out = _mm(at1_lo.reshape(NPADH, D), at1_hi.reshape(NPADH, D), deg_s, s1, Wl_t1, Wr_t1,
                bl_t1.reshape(1, D), relu=False)
    return (s_out, t_out)
